# Initial kernel scaffold; baseline (speedup 1.0000x reference)
import jax
import jax.numpy as jnp
from jax.experimental import pallas as pl


def kernel(x, edge_index, W1l, W1r, b1, W2l, W2r, b2):
    raise NotImplementedError("write your pallas kernel here")


# speedup vs baseline: 3.1521x; 3.1521x over previous
import functools

import jax
import jax.numpy as jnp
from jax import lax
from jax.experimental import pallas as pl
from jax.experimental.pallas import tpu as pltpu
from jax.experimental.pallas import tpu_sc as plsc

_NC = 2
_NS = 16
_NW = _NC * _NS
_G = 128
_K = 8


def _sc_aggregate(nrow, d, nch, with_deg):
  rpt = nrow // _NS
  ngrp = nch // _K

  mesh = plsc.VectorSubcoreMesh(core_axis_name="c", subcore_axis_name="s")
  out_type = [jax.ShapeDtypeStruct((_NC, nrow, d), jnp.float32)]
  scratch = [
      pltpu.VMEM_SHARED((nrow, d), jnp.float32),
      pltpu.VMEM((_K, _G), jnp.int32),
      pltpu.VMEM((_K, _G), jnp.int32),
      pltpu.VMEM((_G, d), jnp.float32),
      pltpu.SemaphoreType.DMA,
  ]
  if with_deg:
    out_type.append(jax.ShapeDtypeStruct((_NC, nrow, d), jnp.float32))

  @functools.partial(
      pl.kernel, mesh=mesh, out_type=out_type, scratch_types=scratch)
  def k(*refs):
    if with_deg:
      (feats_h, edges_h, z_h, aggp_h, degp_h,
       agg_sh, sidx_v, didx_v, gbuf_v, sem) = refs
    else:
      (feats_h, edges_h, z_h, aggp_h,
       agg_sh, sidx_v, didx_v, gbuf_v, sem) = refs
    c = lax.axis_index("c")
    s = lax.axis_index("s")
    wid = c * _NS + s
    ones16 = jnp.ones((16,), jnp.float32)
    pltpu.sync_copy(z_h, agg_sh.at[pl.ds(s * rpt, rpt)])
    plsc.subcore_barrier()

    if with_deg:
      def ofill(i, carry):
        for l in range(d // 16):
          gbuf_v[i, pl.ds(l * 16, 16)] = ones16
        return carry

      lax.fori_loop(0, _G, ofill, 0)

      def dgroup(g, carry):
        pltpu.sync_copy(edges_h.at[1, wid, pl.ds(g * _K, _K)], didx_v)

        def dbody(j, carry2):
          pltpu.sync_copy(gbuf_v, agg_sh.at[didx_v.at[j]], add=True)
          return carry2

        return lax.fori_loop(0, _K, dbody, carry)

      lax.fori_loop(0, ngrp, dgroup, 0)
      plsc.subcore_barrier()
      pltpu.sync_copy(agg_sh.at[pl.ds(s * rpt, rpt)],
                      degp_h.at[c, pl.ds(s * rpt, rpt)])
      plsc.subcore_barrier()
      pltpu.sync_copy(z_h, agg_sh.at[pl.ds(s * rpt, rpt)])
      plsc.subcore_barrier()

    def group(g, carry):
      pltpu.sync_copy(edges_h.at[0, wid, pl.ds(g * _K, _K)], sidx_v)
      pltpu.sync_copy(edges_h.at[1, wid, pl.ds(g * _K, _K)], didx_v)

      def body(j, carry2):
        pltpu.async_copy(feats_h.at[sidx_v.at[j]], gbuf_v, sem).wait()
        pltpu.sync_copy(gbuf_v, agg_sh.at[didx_v.at[j]], add=True)
        return carry2

      return lax.fori_loop(0, _K, body, carry)

    lax.fori_loop(0, ngrp, group, 0)
    plsc.subcore_barrier()
    pltpu.sync_copy(agg_sh.at[pl.ds(s * rpt, rpt)],
                    aggp_h.at[c, pl.ds(s * rpt, rpt)])

  return k


def _tc_layer(n, d_in, d_out, br, relu, softmax):
  grid = (n // br,)
  row = lambda i: (i, 0)
  fixed = lambda i: (0, 0)

  def body(p0, p1, dp0, dp1, x_r, wl, wr, b, o):
    deg = jnp.maximum(dp0[:, 0:1] + dp1[:, 0:1], 1.0)
    agg = (p0[...] + p1[...]) / deg
    y = lax.dot_general(agg, wl[...], (((1,), (1,)), ((), ())),
                        precision=lax.Precision.HIGHEST,
                        preferred_element_type=jnp.float32)
    y = y + lax.dot_general(x_r[...], wr[...], (((1,), (1,)), ((), ())),
                            precision=lax.Precision.HIGHEST,
                            preferred_element_type=jnp.float32)
    y = y + b[...]
    if relu:
      y = jnp.maximum(y, 0.0)
    if softmax:
      m = jnp.max(y, axis=1, keepdims=True)
      z = y - m
      y = z - jnp.log(jnp.sum(jnp.exp(z), axis=1, keepdims=True))
    o[...] = y

  return pl.pallas_call(
      body,
      grid=grid,
      in_specs=[
          pl.BlockSpec((br, d_in), row),
          pl.BlockSpec((br, d_in), row),
          pl.BlockSpec((br, d_in), row),
          pl.BlockSpec((br, d_in), row),
          pl.BlockSpec((br, d_in), row),
          pl.BlockSpec((d_out, d_in), fixed),
          pl.BlockSpec((d_out, d_in), fixed),
          pl.BlockSpec((1, d_out), fixed),
      ],
      out_specs=pl.BlockSpec((br, d_out), row),
      out_shape=jax.ShapeDtypeStruct((n, d_out), jnp.float32),
  )


def kernel(x, edge_index, W1l, W1r, b1, W2l, W2r, b2):
  n, d_in = x.shape
  e = edge_index.shape[1]
  d_h = W1l.shape[0]
  d_out = W2l.shape[0]
  per = e // _NW
  nch = -(-per // (_G * _K)) * _K
  pad = nch * _G - per
  nrow = -(-n // (_NS * 8)) * _NS * 8
  rpt = nrow // _NS

  src = edge_index[0].astype(jnp.int32).reshape(_NW, per)
  dst = edge_index[1].astype(jnp.int32).reshape(_NW, per)
  srcm = jnp.pad(src, ((0, 0), (0, pad))).reshape(_NW, nch, _G)
  dstm = jnp.pad(dst, ((0, 0), (0, pad)),
                 constant_values=n).reshape(_NW, nch, _G)
  edges = jnp.stack([srcm, dstm])
  z = jnp.zeros((rpt, d_in), jnp.float32)

  aggp1, degp = _sc_aggregate(nrow, d_in, nch, True)(x, edges, z)
  h = _tc_layer(n, d_in, d_h, 1000, True, False)(
      aggp1[0, :n], aggp1[1, :n], degp[0, :n], degp[1, :n], x,
      W1l, W1r, b1.reshape(1, d_h))
  (aggp2,) = _sc_aggregate(nrow, d_h, nch, False)(h, edges, z)
  out = _tc_layer(n, d_h, d_out, 1000, False, True)(
      aggp2[0, :n], aggp2[1, :n], degp[0, :n], degp[1, :n], h,
      W2l, W2r, b2.reshape(1, d_out))
  return out


# speedup vs baseline: 3.4832x; 1.1050x over previous
import functools

import jax
import jax.numpy as jnp
from jax import lax
from jax.experimental import pallas as pl
from jax.experimental.pallas import tpu as pltpu
from jax.experimental.pallas import tpu_sc as plsc

_NC = 2
_NS = 16
_NW = _NC * _NS
_G = 128
_K = 8


def _sc_aggregate(nrow, d, nch, with_deg):
  rpt = nrow // _NS
  ngrp = nch // _K

  mesh = plsc.VectorSubcoreMesh(core_axis_name="c", subcore_axis_name="s")
  out_type = [jax.ShapeDtypeStruct((_NC, nrow, d), jnp.float32)]
  scratch = [
      pltpu.VMEM_SHARED((nrow, d), jnp.float32),
      pltpu.VMEM((_K, _G), jnp.int32),
      pltpu.VMEM((_K, _G), jnp.int32),
      pltpu.VMEM((_G, d), jnp.float32),
      pltpu.VMEM((_G, d), jnp.float32),
      pltpu.SemaphoreType.DMA,
      pltpu.SemaphoreType.DMA,
      pltpu.SemaphoreType.DMA,
      pltpu.SemaphoreType.DMA,
  ]
  if with_deg:
    out_type.append(jax.ShapeDtypeStruct((_NC, nrow, d), jnp.float32))

  @functools.partial(
      pl.kernel, mesh=mesh, out_type=out_type, scratch_types=scratch)
  def k(*refs):
    if with_deg:
      (feats_h, edges_h, z_h, aggp_h, degp_h,
       agg_sh, sidx_v, didx_v, gbuf_v, gbuf2_v,
       semg0, semg1, sems0, sems1) = refs
    else:
      (feats_h, edges_h, z_h, aggp_h,
       agg_sh, sidx_v, didx_v, gbuf_v, gbuf2_v,
       semg0, semg1, sems0, sems1) = refs
    c = lax.axis_index("c")
    s = lax.axis_index("s")
    wid = c * _NS + s
    ones16 = jnp.ones((16,), jnp.float32)
    pltpu.sync_copy(z_h, agg_sh.at[pl.ds(s * rpt, rpt)])
    plsc.subcore_barrier()

    if with_deg:
      def ofill(i, carry):
        for l in range(d // 16):
          gbuf_v[i, pl.ds(l * 16, 16)] = ones16
        return carry

      lax.fori_loop(0, _G, ofill, 0)

      def dgroup(g, carry):
        pltpu.sync_copy(edges_h.at[1, wid, pl.ds(g * _K, _K)], didx_v)
        descs = [
            pltpu.async_copy(gbuf_v, agg_sh.at[didx_v.at[j]], semg0,
                             add=True)
            for j in range(_K)
        ]
        for de in descs:
          de.wait()
        return carry

      lax.fori_loop(0, ngrp, dgroup, 0)
      plsc.subcore_barrier()
      pltpu.sync_copy(agg_sh.at[pl.ds(s * rpt, rpt)],
                      degp_h.at[c, pl.ds(s * rpt, rpt)])
      plsc.subcore_barrier()
      pltpu.sync_copy(z_h, agg_sh.at[pl.ds(s * rpt, rpt)])
      plsc.subcore_barrier()

    bufs = (gbuf_v, gbuf2_v)
    gsems = (semg0, semg1)
    ssems = (sems0, sems1)

    def group(g, carry):
      pltpu.sync_copy(edges_h.at[0, wid, pl.ds(g * _K, _K)], sidx_v)
      pltpu.sync_copy(edges_h.at[1, wid, pl.ds(g * _K, _K)], didx_v)
      pend = pltpu.async_copy(feats_h.at[sidx_v.at[0]], bufs[0], gsems[0])
      scat = [None, None]
      for j in range(_K):
        b = j % 2
        nxt = None
        if j + 1 < _K:
          nb = 1 - b
          if scat[nb] is not None:
            scat[nb].wait()
          nxt = pltpu.async_copy(feats_h.at[sidx_v.at[j + 1]], bufs[nb],
                                 gsems[nb])
        pend.wait()
        scat[b] = pltpu.async_copy(bufs[b], agg_sh.at[didx_v.at[j]],
                                   ssems[b], add=True)
        pend = nxt
      for de in scat:
        if de is not None:
          de.wait()
      return carry

    lax.fori_loop(0, ngrp, group, 0)
    plsc.subcore_barrier()
    pltpu.sync_copy(agg_sh.at[pl.ds(s * rpt, rpt)],
                    aggp_h.at[c, pl.ds(s * rpt, rpt)])

  return k


def _tc_layer(n, d_in, d_out, br, relu, softmax):
  grid = (n // br,)
  row = lambda i: (i, 0)
  fixed = lambda i: (0, 0)

  def body(p0, p1, dp0, dp1, x_r, wl, wr, b, o):
    deg = jnp.maximum(dp0[:, 0:1] + dp1[:, 0:1], 1.0)
    agg = (p0[...] + p1[...]) / deg
    y = lax.dot_general(agg, wl[...], (((1,), (1,)), ((), ())),
                        precision=lax.Precision.HIGHEST,
                        preferred_element_type=jnp.float32)
    y = y + lax.dot_general(x_r[...], wr[...], (((1,), (1,)), ((), ())),
                            precision=lax.Precision.HIGHEST,
                            preferred_element_type=jnp.float32)
    y = y + b[...]
    if relu:
      y = jnp.maximum(y, 0.0)
    if softmax:
      m = jnp.max(y, axis=1, keepdims=True)
      z = y - m
      y = z - jnp.log(jnp.sum(jnp.exp(z), axis=1, keepdims=True))
    o[...] = y

  return pl.pallas_call(
      body,
      grid=grid,
      in_specs=[
          pl.BlockSpec((br, d_in), row),
          pl.BlockSpec((br, d_in), row),
          pl.BlockSpec((br, d_in), row),
          pl.BlockSpec((br, d_in), row),
          pl.BlockSpec((br, d_in), row),
          pl.BlockSpec((d_out, d_in), fixed),
          pl.BlockSpec((d_out, d_in), fixed),
          pl.BlockSpec((1, d_out), fixed),
      ],
      out_specs=pl.BlockSpec((br, d_out), row),
      out_shape=jax.ShapeDtypeStruct((n, d_out), jnp.float32),
  )


def kernel(x, edge_index, W1l, W1r, b1, W2l, W2r, b2):
  n, d_in = x.shape
  e = edge_index.shape[1]
  d_h = W1l.shape[0]
  d_out = W2l.shape[0]
  per = e // _NW
  nch = -(-per // (_G * _K)) * _K
  pad = nch * _G - per
  nrow = -(-n // (_NS * 8)) * _NS * 8
  rpt = nrow // _NS

  src = edge_index[0].astype(jnp.int32).reshape(_NW, per)
  dst = edge_index[1].astype(jnp.int32).reshape(_NW, per)
  srcm = jnp.pad(src, ((0, 0), (0, pad))).reshape(_NW, nch, _G)
  dstm = jnp.pad(dst, ((0, 0), (0, pad)),
                 constant_values=n).reshape(_NW, nch, _G)
  edges = jnp.stack([srcm, dstm])
  z = jnp.zeros((rpt, d_in), jnp.float32)

  aggp1, degp = _sc_aggregate(nrow, d_in, nch, True)(x, edges, z)
  h = _tc_layer(n, d_in, d_h, 1000, True, False)(
      aggp1[0, :n], aggp1[1, :n], degp[0, :n], degp[1, :n], x,
      W1l, W1r, b1.reshape(1, d_h))
  (aggp2,) = _sc_aggregate(nrow, d_h, nch, False)(h, edges, z)
  out = _tc_layer(n, d_h, d_out, 1000, False, True)(
      aggp2[0, :n], aggp2[1, :n], degp[0, :n], degp[1, :n], h,
      W2l, W2r, b2.reshape(1, d_out))
  return out
